# baseline (speedup 1.0000x reference)
import functools

import jax
import jax.numpy as jnp
from jax import lax
from jax.experimental import pallas as pl
from jax.experimental.pallas import tpu as pltpu
from jax.experimental.pallas import tpu_sc as plsc

N = 10000
E = 320000
D = 128
H = 128
C = 40
G = 128

NC = 2
NS = 16
NW = NC * NS
EPW = E // NW
K = 125
NITER = -(-EPW // K)
EPAD = NITER * K
PADK = EPAD - EPW
RPT = 624
TAIL = N - NS * RPT

RB = 1000
GRID = N // RB

_f32 = jnp.float32


def _slab_copy(src_hbm, dst_sh, s):
    off = pl.multiple_of(s * RPT, 8)
    pltpu.sync_copy(src_hbm.at[pl.ds(off, RPT)], dst_sh.at[pl.ds(off, RPT)])

    @pl.when(s == NS - 1)
    def _():
        pltpu.sync_copy(src_hbm.at[pl.ds(NS * RPT, TAIL)],
                        dst_sh.at[pl.ds(NS * RPT, TAIL)])


def _slab_out(src_sh, out_hbm, c, s):
    off = pl.multiple_of(s * RPT, 8)
    pltpu.sync_copy(src_sh.at[pl.ds(off, RPT)], out_hbm.at[c, pl.ds(off, RPT)])

    @pl.when(s == NS - 1)
    def _():
        pltpu.sync_copy(src_sh.at[pl.ds(NS * RPT, TAIL)],
                        out_hbm.at[c, pl.ds(NS * RPT, TAIL)])


@functools.cache
def _sc_kernels():
    mesh = plsc.VectorSubcoreMesh(core_axis_name="c", subcore_axis_name="s",
                                  num_cores=NC, num_subcores=NS)

    @functools.partial(
        pl.kernel,
        out_type=jax.ShapeDtypeStruct((NC, N, H), _f32),
        mesh=mesh,
        scratch_types=[
            pltpu.VMEM((NITER, K), jnp.int32),
            pltpu.VMEM((K, H), _f32),
            pltpu.VMEM((K, H), _f32),
            pltpu.VMEM_SHARED((N, H), _f32),
        ],
    )
    def _deg_sc(col_hbm, ones_hbm, tail_hbm, zeros_hbm, out_hbm,
                col_v, ones_v, tail_v, acc_sh):
        c = lax.axis_index("c")
        s = lax.axis_index("s")
        wid = c * NS + s
        pltpu.sync_copy(col_hbm.at[wid], col_v)
        pltpu.sync_copy(ones_hbm, ones_v)
        pltpu.sync_copy(tail_hbm, tail_v)
        _slab_copy(zeros_hbm, acc_sh, s)
        plsc.subcore_barrier()

        def body(j, _):
            pltpu.sync_copy(ones_v, acc_sh.at[col_v.at[j]], add=True)
            return 0

        lax.fori_loop(0, NITER - 1, body, 0)
        pltpu.sync_copy(tail_v, acc_sh.at[col_v.at[NITER - 1]], add=True)
        plsc.subcore_barrier()
        _slab_out(acc_sh, out_hbm, c, s)

    @functools.partial(
        pl.kernel,
        out_type=jax.ShapeDtypeStruct((NC, N, H), _f32),
        mesh=mesh,
        scratch_types=[
            pltpu.VMEM((NITER, K), jnp.int32),
            pltpu.VMEM((NITER, K), jnp.int32),
            pltpu.VMEM((K, H), _f32),
            pltpu.VMEM_SHARED((N, H), _f32),
        ],
    )
    def _agg_sc(row_hbm, col_hbm, y_hbm, zeros_hbm, out_hbm,
                row_v, col_v, buf, acc_sh):
        c = lax.axis_index("c")
        s = lax.axis_index("s")
        wid = c * NS + s
        pltpu.sync_copy(row_hbm.at[wid], row_v)
        pltpu.sync_copy(col_hbm.at[wid], col_v)
        _slab_copy(zeros_hbm, acc_sh, s)
        plsc.subcore_barrier()

        def body(j, _):
            pltpu.sync_copy(y_hbm.at[row_v.at[j]], buf)
            pltpu.sync_copy(buf, acc_sh.at[col_v.at[j]], add=True)
            return 0

        lax.fori_loop(0, NITER, body, 0)
        plsc.subcore_barrier()
        _slab_out(acc_sh, out_hbm, c, s)

    return _deg_sc, _agg_sc


def _dinv_block(degp):
    deg = degp[0, :, 0:1] + degp[1, :, 0:1] + 1.0
    return lax.rsqrt(deg)


def _mm_body(x_ref, w_ref, y_ref):
    y_ref[...] = jnp.dot(x_ref[...], w_ref[...], preferred_element_type=_f32)


_mm = pl.pallas_call(
    _mm_body,
    grid=(GRID,),
    in_specs=[
        pl.BlockSpec((RB, D), lambda i: (i, 0)),
        pl.BlockSpec((D, H), lambda i: (0, 0)),
    ],
    out_specs=pl.BlockSpec((RB, H), lambda i: (i, 0)),
    out_shape=jax.ShapeDtypeStruct((N, H), _f32),
)


def _scale_body(xw_ref, degp_ref, y_ref):
    y_ref[...] = xw_ref[...] * _dinv_block(degp_ref[...])


_scale = pl.pallas_call(
    _scale_body,
    grid=(GRID,),
    in_specs=[
        pl.BlockSpec((RB, H), lambda i: (i, 0)),
        pl.BlockSpec((NC, RB, H), lambda i: (0, i, 0)),
    ],
    out_specs=pl.BlockSpec((RB, H), lambda i: (i, 0)),
    out_shape=jax.ShapeDtypeStruct((N, H), _f32),
)


def _fuse_body(zp_ref, y_ref, degp_ref, b_ref, w_ref, o_ref):
    dinv = _dinv_block(degp_ref[...])
    z = zp_ref[0] + zp_ref[1] + y_ref[...]
    h = jnp.maximum(z * dinv + b_ref[...], 0.0)
    o_ref[...] = jnp.dot(h, w_ref[...], preferred_element_type=_f32) * dinv


_fuse = pl.pallas_call(
    _fuse_body,
    grid=(GRID,),
    in_specs=[
        pl.BlockSpec((NC, RB, H), lambda i: (0, i, 0)),
        pl.BlockSpec((RB, H), lambda i: (i, 0)),
        pl.BlockSpec((NC, RB, H), lambda i: (0, i, 0)),
        pl.BlockSpec((1, H), lambda i: (0, 0)),
        pl.BlockSpec((H, H), lambda i: (0, 0)),
    ],
    out_specs=pl.BlockSpec((RB, H), lambda i: (i, 0)),
    out_shape=jax.ShapeDtypeStruct((N, H), _f32),
)


def _pool_body(zp_ref, y_ref, degp_ref, b_ref, batch_ref, wlin_ref, blin_ref,
               o_ref, pooled_acc, cnt_acc):
    i = pl.program_id(0)

    @pl.when(i == 0)
    def _():
        pooled_acc[...] = jnp.zeros((G, H), _f32)
        cnt_acc[...] = jnp.zeros((G, H), _f32)

    dinv = _dinv_block(degp_ref[...])
    z = zp_ref[0] + zp_ref[1] + y_ref[...]
    h = jnp.maximum(z * dinv + b_ref[...], 0.0)

    g_iota = lax.broadcasted_iota(jnp.int32, (RB, G), 1)
    onehot = (batch_ref[...] == g_iota).astype(_f32)
    dn = (((0,), (0,)), ((), ()))
    pooled_acc[...] += lax.dot_general(onehot, h, dn,
                                       preferred_element_type=_f32)
    cnt_acc[...] += lax.dot_general(onehot, jnp.ones((RB, H), _f32), dn,
                                    preferred_element_type=_f32)

    @pl.when(i == GRID - 1)
    def _():
        pooled = pooled_acc[...] / jnp.maximum(cnt_acc[...], 1.0)
        o_ref[...] = jnp.dot(pooled, wlin_ref[...],
                             preferred_element_type=_f32) + blin_ref[...]


_pool = pl.pallas_call(
    _pool_body,
    grid=(GRID,),
    in_specs=[
        pl.BlockSpec((NC, RB, H), lambda i: (0, i, 0)),
        pl.BlockSpec((RB, H), lambda i: (i, 0)),
        pl.BlockSpec((NC, RB, H), lambda i: (0, i, 0)),
        pl.BlockSpec((1, H), lambda i: (0, 0)),
        pl.BlockSpec((RB, 1), lambda i: (i, 0)),
        pl.BlockSpec((H, 128), lambda i: (0, 0)),
        pl.BlockSpec((1, 128), lambda i: (0, 0)),
    ],
    out_specs=pl.BlockSpec((G, 128), lambda i: (0, 0)),
    out_shape=jax.ShapeDtypeStruct((G, 128), _f32),
    scratch_shapes=[
        pltpu.VMEM((G, H), _f32),
        pltpu.VMEM((G, H), _f32),
    ],
)


def kernel(x, edge_index, batch, W1, b1, W2, b2, Wlin, blin):
    row2 = edge_index[0].astype(jnp.int32).reshape(NW, EPW)
    col2 = edge_index[1].astype(jnp.int32).reshape(NW, EPW)
    if PADK:
        row2 = jnp.pad(row2, ((0, 0), (0, PADK)), constant_values=N)
        pad_dst = jnp.broadcast_to(jnp.arange(PADK, dtype=jnp.int32),
                                   (NW, PADK))
        col2 = jnp.concatenate([col2, pad_dst], axis=1)
    row = row2.reshape(NW, NITER, K)
    col = col2.reshape(NW, NITER, K)

    onesH = jnp.ones((K, H), _f32)
    tailH = jnp.concatenate([jnp.ones((K - PADK, H), _f32),
                             jnp.zeros((PADK, H), _f32)])
    zerosH = jnp.zeros((N, H), _f32)

    _deg_sc, _agg_sc = _sc_kernels()
    xw = _mm(x, W1)
    degp = _deg_sc(col, onesH, tailH, zerosH)
    y1 = _scale(xw, degp)
    z1 = _agg_sc(row, col, jnp.pad(y1, ((0, 8), (0, 0))) if PADK else y1,
                 zerosH)
    y2 = _fuse(z1, y1, degp, b1.reshape(1, H), W2)
    z2 = _agg_sc(row, col, jnp.pad(y2, ((0, 8), (0, 0))) if PADK else y2,
                 zerosH)

    wlin_pad = jnp.pad(Wlin, ((0, 0), (0, 128 - C)))
    blin_pad = jnp.pad(blin, (0, 128 - C)).reshape(1, 128)
    out = _pool(z2, y2, degp, b2.reshape(1, H),
                batch.astype(jnp.int32).reshape(N, 1), wlin_pad, blin_pad)
    return out[:, :C]


# speedup vs baseline: 1.1048x; 1.1048x over previous
import functools

import jax
import jax.numpy as jnp
from jax import lax
from jax.experimental import pallas as pl
from jax.experimental.pallas import tpu as pltpu
from jax.experimental.pallas import tpu_sc as plsc

N = 10000
E = 320000
D = 128
H = 128
C = 40
G = 128

NC = 2
NS = 16
NW = NC * NS
EPW = E // NW
K = 125
VB = 16
NV = EPW // VB
NITER = -(-EPW // K)
EPAD = NITER * K
PADK = EPAD - EPW
RPT = 624
TAIL = N - NS * RPT

RB = 1000
GRID = N // RB

_f32 = jnp.float32


def _slab_copy(src_hbm, dst_sh, s):
    off = pl.multiple_of(s * RPT, 8)
    pltpu.sync_copy(src_hbm.at[pl.ds(off, RPT)], dst_sh.at[pl.ds(off, RPT)])

    @pl.when(s == NS - 1)
    def _():
        pltpu.sync_copy(src_hbm.at[pl.ds(NS * RPT, TAIL)],
                        dst_sh.at[pl.ds(NS * RPT, TAIL)])


def _slab_out(src_sh, out_hbm, c, s):
    off = pl.multiple_of(s * RPT, 8)
    pltpu.sync_copy(src_sh.at[pl.ds(off, RPT)], out_hbm.at[c, pl.ds(off, RPT)])

    @pl.when(s == NS - 1)
    def _():
        pltpu.sync_copy(src_sh.at[pl.ds(NS * RPT, TAIL)],
                        out_hbm.at[c, pl.ds(NS * RPT, TAIL)])


@functools.cache
def _sc_kernels():
    mesh = plsc.VectorSubcoreMesh(core_axis_name="c", subcore_axis_name="s",
                                  num_cores=NC, num_subcores=NS)

    @functools.partial(
        pl.kernel,
        out_type=jax.ShapeDtypeStruct((NW, N), jnp.int32),
        mesh=mesh,
        compiler_params=pltpu.CompilerParams(needs_layout_passes=False),
        scratch_types=[
            pltpu.VMEM((NV, VB), jnp.int32),
            pltpu.VMEM((N,), jnp.int32),
        ],
    )
    def _deg_sc(col_hbm, out_hbm, col_v, hist_v):
        c = lax.axis_index("c")
        s = lax.axis_index("s")
        wid = c * NS + s
        pltpu.sync_copy(col_hbm.at[wid], col_v)

        def zbody(j, _):
            hist_v[pl.ds(pl.multiple_of(j * VB, VB), VB)] = jnp.zeros(
                (VB,), jnp.int32)
            return 0

        lax.fori_loop(0, N // VB, zbody, 0)

        def body(j, _):
            xv = col_v[j]
            cnt, last = plsc.scan_count(xv)
            plsc.addupdate_scatter(hist_v, [xv], cnt, mask=last)
            return 0

        lax.fori_loop(0, NV, body, 0)
        pltpu.sync_copy(hist_v, out_hbm.at[wid])

    @functools.partial(
        pl.kernel,
        out_type=jax.ShapeDtypeStruct((NC, N, H), _f32),
        mesh=mesh,
        scratch_types=[
            pltpu.VMEM((NITER, K), jnp.int32),
            pltpu.VMEM((NITER, K), jnp.int32),
            pltpu.VMEM((K, H), _f32),
            pltpu.VMEM_SHARED((N, H), _f32),
        ],
    )
    def _agg_sc(row_hbm, col_hbm, y_hbm, zeros_hbm, out_hbm,
                row_v, col_v, buf, acc_sh):
        c = lax.axis_index("c")
        s = lax.axis_index("s")
        wid = c * NS + s
        pltpu.sync_copy(row_hbm.at[wid], row_v)
        pltpu.sync_copy(col_hbm.at[wid], col_v)
        _slab_copy(zeros_hbm, acc_sh, s)
        plsc.subcore_barrier()

        def body(j, _):
            pltpu.sync_copy(y_hbm.at[row_v.at[j]], buf)
            pltpu.sync_copy(buf, acc_sh.at[col_v.at[j]], add=True)
            return 0

        lax.fori_loop(0, NITER, body, 0)
        plsc.subcore_barrier()
        _slab_out(acc_sh, out_hbm, c, s)

    return _deg_sc, _agg_sc


def _dinv_block(degp):
    deg = lax.dot_general(degp.astype(_f32), jnp.ones((NW, 1), _f32),
                          (((0,), (0,)), ((), ())),
                          preferred_element_type=_f32) + 1.0
    return lax.rsqrt(deg)


def _mm_body(x_ref, w_ref, y_ref):
    y_ref[...] = jnp.dot(x_ref[...], w_ref[...], preferred_element_type=_f32)


_mm = pl.pallas_call(
    _mm_body,
    grid=(GRID,),
    in_specs=[
        pl.BlockSpec((RB, D), lambda i: (i, 0)),
        pl.BlockSpec((D, H), lambda i: (0, 0)),
    ],
    out_specs=pl.BlockSpec((RB, H), lambda i: (i, 0)),
    out_shape=jax.ShapeDtypeStruct((N, H), _f32),
)


def _scale_body(xw_ref, degp_ref, y_ref):
    y_ref[...] = xw_ref[...] * _dinv_block(degp_ref[0])


_scale = pl.pallas_call(
    _scale_body,
    grid=(GRID,),
    in_specs=[
        pl.BlockSpec((RB, H), lambda i: (i, 0)),
        pl.BlockSpec((1, NW, RB), lambda i: (i, 0, 0)),
    ],
    out_specs=pl.BlockSpec((RB, H), lambda i: (i, 0)),
    out_shape=jax.ShapeDtypeStruct((N, H), _f32),
)


def _fuse_body(zp_ref, y_ref, degp_ref, b_ref, w_ref, o_ref):
    dinv = _dinv_block(degp_ref[0])
    z = zp_ref[0] + zp_ref[1] + y_ref[...]
    h = jnp.maximum(z * dinv + b_ref[...], 0.0)
    o_ref[...] = jnp.dot(h, w_ref[...], preferred_element_type=_f32) * dinv


_fuse = pl.pallas_call(
    _fuse_body,
    grid=(GRID,),
    in_specs=[
        pl.BlockSpec((NC, RB, H), lambda i: (0, i, 0)),
        pl.BlockSpec((RB, H), lambda i: (i, 0)),
        pl.BlockSpec((1, NW, RB), lambda i: (i, 0, 0)),
        pl.BlockSpec((1, H), lambda i: (0, 0)),
        pl.BlockSpec((H, H), lambda i: (0, 0)),
    ],
    out_specs=pl.BlockSpec((RB, H), lambda i: (i, 0)),
    out_shape=jax.ShapeDtypeStruct((N, H), _f32),
)


def _pool_body(zp_ref, y_ref, degp_ref, b_ref, batch_ref, wlin_ref, blin_ref,
               o_ref, pooled_acc, cnt_acc):
    i = pl.program_id(0)

    @pl.when(i == 0)
    def _():
        pooled_acc[...] = jnp.zeros((G, H), _f32)
        cnt_acc[...] = jnp.zeros((G, H), _f32)

    dinv = _dinv_block(degp_ref[0])
    z = zp_ref[0] + zp_ref[1] + y_ref[...]
    h = jnp.maximum(z * dinv + b_ref[...], 0.0)

    g_iota = lax.broadcasted_iota(jnp.int32, (RB, G), 1)
    onehot = (batch_ref[...] == g_iota).astype(_f32)
    dn = (((0,), (0,)), ((), ()))
    pooled_acc[...] += lax.dot_general(onehot, h, dn,
                                       preferred_element_type=_f32)
    cnt_acc[...] += lax.dot_general(onehot, jnp.ones((RB, H), _f32), dn,
                                    preferred_element_type=_f32)

    @pl.when(i == GRID - 1)
    def _():
        pooled = pooled_acc[...] / jnp.maximum(cnt_acc[...], 1.0)
        o_ref[...] = jnp.dot(pooled, wlin_ref[...],
                             preferred_element_type=_f32) + blin_ref[...]


_pool = pl.pallas_call(
    _pool_body,
    grid=(GRID,),
    in_specs=[
        pl.BlockSpec((NC, RB, H), lambda i: (0, i, 0)),
        pl.BlockSpec((RB, H), lambda i: (i, 0)),
        pl.BlockSpec((1, NW, RB), lambda i: (i, 0, 0)),
        pl.BlockSpec((1, H), lambda i: (0, 0)),
        pl.BlockSpec((RB, 1), lambda i: (i, 0)),
        pl.BlockSpec((H, 128), lambda i: (0, 0)),
        pl.BlockSpec((1, 128), lambda i: (0, 0)),
    ],
    out_specs=pl.BlockSpec((G, 128), lambda i: (0, 0)),
    out_shape=jax.ShapeDtypeStruct((G, 128), _f32),
    scratch_shapes=[
        pltpu.VMEM((G, H), _f32),
        pltpu.VMEM((G, H), _f32),
    ],
)


def kernel(x, edge_index, batch, W1, b1, W2, b2, Wlin, blin):
    row2 = edge_index[0].astype(jnp.int32).reshape(NW, EPW)
    col2 = edge_index[1].astype(jnp.int32).reshape(NW, EPW)
    if PADK:
        row2 = jnp.pad(row2, ((0, 0), (0, PADK)), constant_values=N)
        pad_dst = jnp.broadcast_to(jnp.arange(PADK, dtype=jnp.int32),
                                   (NW, PADK))
        col2 = jnp.concatenate([col2, pad_dst], axis=1)
    row = row2.reshape(NW, NITER, K)
    col = col2.reshape(NW, NITER, K)

    zerosH = jnp.zeros((N, H), _f32)

    _deg_sc, _agg_sc = _sc_kernels()
    xw = _mm(x, W1)
    degp = _deg_sc(col2.reshape(NW, NV, VB))
    degp = degp.reshape(NW, GRID, RB).transpose(1, 0, 2)
    y1 = _scale(xw, degp)
    z1 = _agg_sc(row, col, jnp.pad(y1, ((0, 8), (0, 0))) if PADK else y1,
                 zerosH)
    y2 = _fuse(z1, y1, degp, b1.reshape(1, H), W2)
    z2 = _agg_sc(row, col, jnp.pad(y2, ((0, 8), (0, 0))) if PADK else y2,
                 zerosH)

    wlin_pad = jnp.pad(Wlin, ((0, 0), (0, 128 - C)))
    blin_pad = jnp.pad(blin, (0, 128 - C)).reshape(1, 128)
    out = _pool(z2, y2, degp, b2.reshape(1, H),
                batch.astype(jnp.int32).reshape(N, 1), wlin_pad, blin_pad)
    return out[:, :C]
